# baseline (speedup 1.0000x reference)
import functools

import jax
import jax.numpy as jnp
from jax import lax
from jax.experimental import pallas as pl
from jax.experimental.pallas import tpu as pltpu
from jax.experimental.pallas import tpu_sc as plsc

SIGLEN = 16384
EPS = 0.005
BASE_STEP_SIZE = 0.002
MAX_ITERATIONS = 7
BATCH = 4096
NUM_PEAKS = 3

NPAIR = BATCH * NUM_PEAKS
NWORKERS = 32
PW = NPAIR // NWORKERS
RW = BATCH // NWORKERS
NCH = PW // 16
NQ = 3
WIN = NQ * 128
WOFF = EPS + 0.0015
C0MAX = SIGLEN - WIN


def _refine_body(pred_hbm, sig_hbm, out_hbm, pos_v, c0_v, ri_v, r8_v, n3_v,
                 buf0, buf1, dummy_hbm, sem0, sem1):
    c = lax.axis_index("c")
    s = lax.axis_index("s")
    wid = s * 2 + c
    base0 = wid * PW

    pltpu.sync_copy(pred_hbm.at[pl.ds(base0, PW)], pos_v)

    def fire(k16, buf, sem):
        c0g = c0_v[pl.ds(k16, 16)]
        r8g = r8_v[pl.ds(k16, 16)]
        n3g = n3_v[pl.ds(k16, 16)]
        for t in range(16):
            r8 = pl.multiple_of(r8g[t], 8)
            c0 = pl.multiple_of(c0g[t], 128)
            for q in range(NQ - 1):
                pltpu.async_copy(
                    sig_hbm.at[pl.ds(r8, 8), pl.ds(c0 + q * 128, 128)],
                    buf.at[t, q], sem)
            n3 = n3g[t]

            @pl.when(n3 != 0)
            def _():
                pltpu.async_copy(
                    sig_hbm.at[pl.ds(r8, 8),
                               pl.ds(c0 + (NQ - 1) * 128, 128)],
                    buf.at[t, NQ - 1], sem)

    lanes = lax.iota(jnp.int32, 16)
    for k in range(NCH):
        pos = pos_v[pl.ds(k * 16, 16)]
        p = jnp.clip(pos, EPS, 1.0 - EPS)
        cmin_f = jnp.maximum((p - WOFF) * (SIGLEN - 1), 0.0)
        cmin = jnp.maximum(cmin_f.astype(jnp.int32) - 1, 0)
        c0 = jnp.minimum((cmin >> 7) << 7, C0MAX)
        c0_v[pl.ds(k * 16, 16)] = c0
        cmax = ((p + WOFF) * (SIGLEN - 1)).astype(jnp.int32) + 3
        n3_v[pl.ds(k * 16, 16)] = jnp.where(cmax - c0 > 255, 1, 0)
        pair = k * 16 + lanes
        rloc = (pair * 21846) >> 16
        ri_v[pl.ds(k * 16, 16)] = rloc & 7
        r8_v[pl.ds(k * 16, 16)] = wid * RW + (rloc & ~7)
        if k == 0:
            fire(0, buf0, sem0)
        if k == 1:
            fire(16, buf1, sem1)

    def drain(k16, buf, sem):
        pltpu.make_async_copy(
            dummy_hbm.at[:, pl.ds(0, 2)], buf.at[:, pl.ds(0, 2)],
            sem).wait()
        cnt3 = jnp.sum(n3_v[pl.ds(k16, 16)])

        def w(_, carry):
            pltpu.make_async_copy(
                dummy_hbm.at[0, 0], buf.at[0, NQ - 1], sem).wait()
            return carry

        lax.fori_loop(0, cnt3, w, 0)

    def compute(k16, buf):
        pos0 = pos_v[pl.ds(k16, 16)]
        c0 = c0_v[pl.ds(k16, 16)]
        ri = ri_v[pl.ds(k16, 16)]

        def iter_body(_, pos):
            p = jnp.clip(pos, EPS, 1.0 - EPS)
            vals = []
            for d in (-EPS, 0.0, EPS):
                ps = jnp.clip((p + d) * (SIGLEN - 1), 0.0,
                              float(SIGLEN - 1))
                il = ps.astype(jnp.int32)
                wr = ps - il.astype(jnp.float32)
                ir = jnp.minimum(il + 1, SIGLEN - 1)
                dl = il - c0
                dr = ir - c0
                vl = plsc.load_gather(buf, [lanes, dl >> 7, ri, dl & 127])
                vr = plsc.load_gather(buf, [lanes, dr >> 7, ri, dr & 127])
                vals.append((1.0 - wr) * vl + wr * vr)
            v_left, v_c, v_right = vals
            grad = (v_right - v_left) / (2 * EPS)
            curv = (v_right + v_left - 2.0 * v_c) / (EPS * EPS)
            curv = jnp.clip(curv, -1000.0, 1000.0)
            step = -grad / (jnp.abs(curv) + 1e-6)
            step = jnp.clip(step, -0.1, 0.1)
            return jnp.clip(pos + BASE_STEP_SIZE * step, 0.0, 1.0)

        pos_v[pl.ds(k16, 16)] = lax.fori_loop(
            0, MAX_ITERATIONS, iter_body, pos0)

    def outer(m, carry):
        k0 = pl.multiple_of(m * 32, 8)
        drain(k0, buf0, sem0)
        compute(k0, buf0)

        @pl.when(m < NCH // 2 - 1)
        def _():
            fire(k0 + 32, buf0, sem0)

        drain(k0 + 16, buf1, sem1)
        compute(k0 + 16, buf1)

        @pl.when(m < NCH // 2 - 1)
        def _():
            fire(k0 + 48, buf1, sem1)

        return carry

    lax.fori_loop(0, NCH // 2, outer, 0)

    pltpu.sync_copy(pos_v, out_hbm.at[pl.ds(base0, PW)])


@jax.jit
def kernel(initial_predictions, signals):
    if signals.ndim == 3:
        signals = signals[:, 0, :]
    preds = initial_predictions.reshape(-1)

    mesh = plsc.VectorSubcoreMesh(core_axis_name="c", subcore_axis_name="s")
    refine = functools.partial(
        pl.kernel,
        mesh=mesh,
        out_type=jax.ShapeDtypeStruct((NPAIR,), jnp.float32),
        scratch_types=[
            pltpu.VMEM((PW,), jnp.float32),
            pltpu.VMEM((PW,), jnp.int32),
            pltpu.VMEM((PW,), jnp.int32),
            pltpu.VMEM((PW,), jnp.int32),
            pltpu.VMEM((PW,), jnp.int32),
            pltpu.VMEM((16, NQ, 8, 128), jnp.float32),
            pltpu.VMEM((16, NQ, 8, 128), jnp.float32),
            pltpu.HBM((16, NQ, 8, 128), jnp.float32),
            pltpu.SemaphoreType.DMA,
            pltpu.SemaphoreType.DMA,
        ],
        compiler_params=pltpu.CompilerParams(
            use_tc_tiling_on_sc=True, needs_layout_passes=False),
    )(_refine_body)

    out = refine(preds, signals)
    return out.reshape(initial_predictions.shape)


# speedup vs baseline: 1.0213x; 1.0213x over previous
import functools

import jax
import jax.numpy as jnp
from jax import lax
from jax.experimental import pallas as pl
from jax.experimental.pallas import tpu as pltpu
from jax.experimental.pallas import tpu_sc as plsc

SIGLEN = 16384
EPS = 0.005
BASE_STEP_SIZE = 0.002
MAX_ITERATIONS = 7
BATCH = 4096
NUM_PEAKS = 3

NPAIR = BATCH * NUM_PEAKS
NWORKERS = 32
PW = NPAIR // NWORKERS
RW = BATCH // NWORKERS
NCH = PW // 16
NQ = 3
WIN = NQ * 128
WOFF = EPS + 0.0015
C0MAX = SIGLEN - WIN


def _refine_body(pred_hbm, sig_hbm, out_hbm, pos_v, c0_v, ri_v, r8_v, n3_v,
                 buf0, buf1, dummy_hbm, sem0, sem1):
    c = lax.axis_index("c")
    s = lax.axis_index("s")
    wid = s * 2 + c
    base0 = wid * PW

    pltpu.sync_copy(pred_hbm.at[pl.ds(base0, PW)], pos_v)

    def fire(k16, buf, sem):
        c0g = c0_v[pl.ds(k16, 16)]
        r8g = r8_v[pl.ds(k16, 16)]
        n3g = n3_v[pl.ds(k16, 16)]
        for t in range(16):
            r8 = pl.multiple_of(r8g[t], 8)
            c0 = pl.multiple_of(c0g[t], 128)
            for q in range(NQ - 1):
                pltpu.async_copy(
                    sig_hbm.at[pl.ds(r8, 8), pl.ds(c0 + q * 128, 128)],
                    buf.at[t, q], sem)
            n3 = n3g[t]

            @pl.when(n3 != 0)
            def _():
                pltpu.async_copy(
                    sig_hbm.at[pl.ds(r8, 8),
                               pl.ds(c0 + (NQ - 1) * 128, 128)],
                    buf.at[t, NQ - 1], sem)

    lanes = lax.iota(jnp.int32, 16)
    for k in range(NCH):
        pos = pos_v[pl.ds(k * 16, 16)]
        p = jnp.clip(pos, EPS, 1.0 - EPS)
        cmin_f = jnp.maximum((p - WOFF) * (SIGLEN - 1), 0.0)
        cmin = jnp.maximum(cmin_f.astype(jnp.int32) - 1, 0)
        c0 = jnp.minimum((cmin >> 7) << 7, C0MAX)
        c0_v[pl.ds(k * 16, 16)] = c0
        cmax = ((p + WOFF) * (SIGLEN - 1)).astype(jnp.int32) + 3
        n3_v[pl.ds(k * 16, 16)] = jnp.where(cmax - c0 > 255, 1, 0)
        pair = k * 16 + lanes
        rloc = (pair * 21846) >> 16
        ri_v[pl.ds(k * 16, 16)] = rloc & 7
        r8_v[pl.ds(k * 16, 16)] = wid * RW + (rloc & ~7)

    def drain(k16, buf, sem):
        pltpu.make_async_copy(
            dummy_hbm.at[:, pl.ds(0, 2)], buf.at[:, pl.ds(0, 2)],
            sem).wait()
        cnt3 = jnp.sum(n3_v[pl.ds(k16, 16)])

        def w(_, carry):
            pltpu.make_async_copy(
                dummy_hbm.at[0, 0], buf.at[0, NQ - 1], sem).wait()
            return carry

        lax.fori_loop(0, cnt3, w, 0)

    def compute(k16, buf):
        pos0 = pos_v[pl.ds(k16, 16)]
        c0 = c0_v[pl.ds(k16, 16)]
        ri = ri_v[pl.ds(k16, 16)]

        def iter_body(_, pos):
            p = jnp.clip(pos, EPS, 1.0 - EPS)
            vals = []
            for d in (-EPS, 0.0, EPS):
                ps = jnp.clip((p + d) * (SIGLEN - 1), 0.0,
                              float(SIGLEN - 1))
                il = ps.astype(jnp.int32)
                wr = ps - il.astype(jnp.float32)
                ir = jnp.minimum(il + 1, SIGLEN - 1)
                dl = il - c0
                dr = ir - c0
                vl = plsc.load_gather(buf, [lanes, dl >> 7, ri, dl & 127])
                vr = plsc.load_gather(buf, [lanes, dr >> 7, ri, dr & 127])
                vals.append((1.0 - wr) * vl + wr * vr)
            v_left, v_c, v_right = vals
            grad = (v_right - v_left) / (2 * EPS)
            curv = (v_right + v_left - 2.0 * v_c) / (EPS * EPS)
            curv = jnp.clip(curv, -1000.0, 1000.0)
            step = -grad / (jnp.abs(curv) + 1e-6)
            step = jnp.clip(step, -0.1, 0.1)
            return jnp.clip(pos + BASE_STEP_SIZE * step, 0.0, 1.0)

        pos_v[pl.ds(k16, 16)] = lax.fori_loop(
            0, MAX_ITERATIONS, iter_body, pos0)

    fire(0, buf0, sem0)

    def outer(m, carry):
        k0 = pl.multiple_of(m * 32, 8)
        fire(k0 + 16, buf1, sem1)
        drain(k0, buf0, sem0)
        compute(k0, buf0)

        @pl.when(m < NCH // 2 - 1)
        def _():
            fire(k0 + 32, buf0, sem0)

        drain(k0 + 16, buf1, sem1)
        compute(k0 + 16, buf1)
        return carry

    lax.fori_loop(0, NCH // 2, outer, 0)

    pltpu.sync_copy(pos_v, out_hbm.at[pl.ds(base0, PW)])


@jax.jit
def kernel(initial_predictions, signals):
    if signals.ndim == 3:
        signals = signals[:, 0, :]
    preds = initial_predictions.reshape(-1)

    mesh = plsc.VectorSubcoreMesh(core_axis_name="c", subcore_axis_name="s")
    refine = functools.partial(
        pl.kernel,
        mesh=mesh,
        out_type=jax.ShapeDtypeStruct((NPAIR,), jnp.float32),
        scratch_types=[
            pltpu.VMEM((PW,), jnp.float32),
            pltpu.VMEM((PW,), jnp.int32),
            pltpu.VMEM((PW,), jnp.int32),
            pltpu.VMEM((PW,), jnp.int32),
            pltpu.VMEM((PW,), jnp.int32),
            pltpu.VMEM((16, NQ, 8, 128), jnp.float32),
            pltpu.VMEM((16, NQ, 8, 128), jnp.float32),
            pltpu.HBM((16, NQ, 8, 128), jnp.float32),
            pltpu.SemaphoreType.DMA,
            pltpu.SemaphoreType.DMA,
        ],
        compiler_params=pltpu.CompilerParams(
            use_tc_tiling_on_sc=True, needs_layout_passes=False),
    )(_refine_body)

    out = refine(preds, signals)
    return out.reshape(initial_predictions.shape)
